# baseline (speedup 1.0000x reference)
import functools

import jax
import jax.numpy as jnp
from jax import lax
from jax.experimental import pallas as pl
from jax.experimental.pallas import tpu as pltpu
from jax.experimental.pallas import tpu_sc as plsc

FEAT_DIM = 128
EMB_DIM = 64
BATCH = 16384
OUT_DIM = FEAT_DIM + EMB_DIM

NC = 2
NS = 16
NW = NC * NS
BPW = BATCH // NW
NCHUNK = 4
C = BPW // NCHUNK

_mesh = plsc.VectorSubcoreMesh(core_axis_name="c", subcore_axis_name="s")


@functools.partial(
    pl.kernel,
    mesh=_mesh,
    out_type=jax.ShapeDtypeStruct((BATCH, OUT_DIM), jnp.float32),
    scratch_types=[
        pltpu.VMEM((NCHUNK, C), jnp.int32),
        pltpu.VMEM((NCHUNK, C, FEAT_DIM), jnp.float32),
        pltpu.VMEM((NCHUNK, C, EMB_DIM), jnp.float32),
        pltpu.SemaphoreType.DMA,
    ],
    compiler_params=pltpu.CompilerParams(use_tc_tiling_on_sc=False),
)
def _encoder(idx_hbm, feat_hbm, emb_hbm, out_hbm, idx_v, feat_v, emb_v, sem):
    wid = lax.axis_index("s") * NC + lax.axis_index("c")
    base = wid * BPW
    pltpu.sync_copy(idx_hbm.at[wid], idx_v)
    copies = []
    for j in range(NCHUNK):
        copies.append(pltpu.async_copy(feat_hbm.at[idx_v.at[j]], feat_v.at[j], sem))
        copies.append(pltpu.async_copy(emb_hbm.at[idx_v.at[j]], emb_v.at[j], sem))
    for cp in copies:
        cp.wait()
    for j in range(NCHUNK):
        row = base + j * C
        pltpu.sync_copy(feat_v.at[j], out_hbm.at[pl.ds(row, C), pl.ds(0, FEAT_DIM)])
        pltpu.sync_copy(emb_v.at[j], out_hbm.at[pl.ds(row, C), pl.ds(FEAT_DIM, EMB_DIM)])


def kernel(indices, features, emb_table):
    idx = indices.astype(jnp.int32).reshape(NW, NCHUNK, C)
    return _encoder(idx, features, emb_table)
